# baseline (speedup 1.0000x reference)
import functools

import jax
import jax.numpy as jnp
from jax import lax
from jax.experimental import pallas as pl
from jax.experimental.pallas import tpu as pltpu
from jax.experimental.pallas import tpu_sc as plsc

N = 10000
D = 128
H = 128
C = 16
E = 320000

NC = 2
NS = 16
NW = NC * NS
EPW = E // NW
CH = 88
NCHP = 114
EPWP = NCHP * CH
NP = 10240
RPT = NP // NS
SL = 4


def _row_block(rows):
    return 1000 if rows % 1000 == 0 else 1024


def _mm_body(x_ref, w_ref, o_ref):
    o_ref[...] = jnp.dot(x_ref[...], w_ref[...],
                         preferred_element_type=jnp.float32)


def _tc_mm(x, w):
    rows, h2 = x.shape[0], w.shape[1]
    mb = _row_block(rows)
    return pl.pallas_call(
        _mm_body,
        grid=(rows // mb,),
        in_specs=[pl.BlockSpec((mb, x.shape[1]), lambda i: (i, 0)),
                  pl.BlockSpec((x.shape[1], h2), lambda i: (0, 0))],
        out_specs=pl.BlockSpec((mb, h2), lambda i: (i, 0)),
        out_shape=jax.ShapeDtypeStruct((rows, h2), jnp.float32),
    )(x, w)


def _combine_mm_body(p_ref, b_ref, w_ref, o_ref):
    h = jnp.maximum(p_ref[0] + p_ref[1] + b_ref[...], 0.0)
    o_ref[...] = jnp.dot(h, w_ref[...], preferred_element_type=jnp.float32)


def _tc_combine_mm(p, b, w):
    rows, f = p.shape[1], p.shape[2]
    h2 = w.shape[1]
    mb = _row_block(rows)
    return pl.pallas_call(
        _combine_mm_body,
        grid=(rows // mb,),
        in_specs=[pl.BlockSpec((2, mb, f), lambda i: (0, i, 0)),
                  pl.BlockSpec((1, f), lambda i: (0, 0)),
                  pl.BlockSpec((f, h2), lambda i: (0, 0))],
        out_specs=pl.BlockSpec((mb, h2), lambda i: (i, 0)),
        out_shape=jax.ShapeDtypeStruct((rows, h2), jnp.float32),
    )(p, b.reshape(1, f), w)


def _softmax_body(p_ref, b_ref, o_ref):
    z = p_ref[0] + p_ref[1] + b_ref[...]
    z = z - jnp.max(z, axis=-1, keepdims=True)
    e = jnp.exp(z)
    o_ref[...] = e / jnp.sum(e, axis=-1, keepdims=True)


def _tc_softmax(p, b):
    rows, f = p.shape[1], p.shape[2]
    mb = _row_block(rows)
    return pl.pallas_call(
        _softmax_body,
        grid=(rows // mb,),
        in_specs=[pl.BlockSpec((2, mb, f), lambda i: (0, i, 0)),
                  pl.BlockSpec((1, f), lambda i: (0, 0))],
        out_specs=pl.BlockSpec((mb, f), lambda i: (i, 0)),
        out_shape=jax.ShapeDtypeStruct((rows, f), jnp.float32),
    )(p, b.reshape(1, f))


def _spmm_body(f, m_hbm, meta_hbm, val_hbm, out_hbm,
               meta0, meta1, meta2, meta3, val0, val1, val2, val3,
               rows0, rows1, rows2, rows3, acc,
               psem0, psem1, psem2, psem3, gsem0, gsem1, gsem2, gsem3,
               ssem0, ssem1, ssem2, ssem3):
    c = lax.axis_index("c")
    s = lax.axis_index("s")
    w = c * NS + s
    base = w * NCHP

    meta = (meta0, meta1, meta2, meta3)
    valv = (val0, val1, val2, val3)
    rows = (rows0, rows1, rows2, rows3)
    psem = (psem0, psem1, psem2, psem3)
    gsem = (gsem0, gsem1, gsem2, gsem3)
    ssem = (ssem0, ssem1, ssem2, ssem3)

    zvec = jnp.zeros((16,), jnp.float32)

    def zrow(r, _):
        for t in range(f // 16):
            rows0[r, pl.ds(16 * t, 16)] = zvec
        return ()

    lax.fori_loop(0, CH, zrow, ())
    for i in range(RPT // CH):
        pltpu.sync_copy(rows0, acc.at[pl.ds(s * RPT + i * CH, CH)])
    rem = RPT - (RPT // CH) * CH
    if rem:
        pltpu.sync_copy(rows0.at[pl.ds(0, rem)],
                        acc.at[pl.ds(s * RPT + (RPT // CH) * CH, rem)])
    plsc.subcore_barrier()

    def start_meta(i, b):
        pltpu.async_copy(meta_hbm.at[base + i], meta[b], psem[b])
        pltpu.async_copy(val_hbm.at[base + i], valv[b], psem[b])

    def wait_meta(i, b):
        pltpu.make_async_copy(meta_hbm.at[base + i], meta[b], psem[b]).wait()
        pltpu.make_async_copy(val_hbm.at[base + i], valv[b], psem[b]).wait()

    def start_gather(b):
        pltpu.async_copy(m_hbm.at[meta[b].at[0]], rows[b], gsem[b])

    def wait_gather(b):
        pltpu.make_async_copy(m_hbm.at[meta[b].at[0]], rows[b],
                              gsem[b]).wait()

    def start_scatter(b):
        pltpu.async_copy(rows[b], acc.at[meta[b].at[1]], ssem[b], add=True)

    def wait_scatter(b):
        pltpu.make_async_copy(rows[b], acc.at[meta[b].at[1]], ssem[b]).wait()

    def step(i, b):
        @pl.when(i >= 2)
        def _():
            wait_scatter((b + 2) % SL)

        @pl.when(i + 2 < NCHP)
        def _():
            start_meta(i + 2, (b + 2) % SL)

        @pl.when(i + 1 < NCHP)
        def _():
            wait_meta(i + 1, (b + 1) % SL)
            start_gather((b + 1) % SL)

        wait_gather(b)

        def scale(g, _):
            vv = valv[b][pl.ds(16 * g, 16)]
            for jj in range(16):
                j = 16 * g + jj
                bv = lax.broadcast(vv[jj], (16,))
                for t in range(f // 16):
                    sl = pl.ds(16 * t, 16)
                    rows[b][j, sl] = rows[b][j, sl] * bv
            return ()

        lax.fori_loop(0, CH // 16, scale, ())

        ntail = CH - (CH // 16) * 16
        if ntail:
            vv = valv[b][pl.ds(CH - 16, 16)]
            for jj in range(16 - ntail, 16):
                j = CH - 16 + jj
                bv = lax.broadcast(vv[jj], (16,))
                for t in range(f // 16):
                    sl = pl.ds(16 * t, 16)
                    rows[b][j, sl] = rows[b][j, sl] * bv

        start_scatter(b)

    start_meta(0, 0)
    start_meta(1, 1)
    wait_meta(0, 0)
    start_gather(0)

    def chunk(i, _):
        for b in range(SL):
            @pl.when(i % SL == b)
            def _(b=b):
                step(i, b)
        return ()

    lax.fori_loop(0, NCHP, chunk, ())
    wait_scatter((NCHP - 2) % SL)
    wait_scatter((NCHP - 1) % SL)
    plsc.subcore_barrier()

    pltpu.sync_copy(acc.at[pl.ds(s * RPT, RPT)],
                    out_hbm.at[c, pl.ds(s * RPT, RPT)])


def _sc_spmm(m, meta, vals):
    f = m.shape[1]
    mesh = plsc.VectorSubcoreMesh(core_axis_name="c", subcore_axis_name="s",
                                  num_cores=NC, num_subcores=NS)
    return pl.kernel(
        functools.partial(_spmm_body, f),
        out_type=jax.ShapeDtypeStruct((2, NP, f), jnp.float32),
        mesh=mesh,
        compiler_params=pltpu.CompilerParams(use_tc_tiling_on_sc=False),
        scratch_types=(
            [pltpu.VMEM((2, CH), jnp.int32) for _ in range(SL)]
            + [pltpu.VMEM((CH,), jnp.float32) for _ in range(SL)]
            + [pltpu.VMEM((CH, f), jnp.float32) for _ in range(SL)]
            + [pltpu.VMEM_SHARED((NP, f), jnp.float32)]
            + [pltpu.SemaphoreType.DMA for _ in range(3 * SL)]
        ),
    )(m, meta, vals)


def _pack_meta(src, dst, vals):
    pad = EPWP - EPW
    srcp = jnp.pad(src.reshape(NW, EPW), ((0, 0), (0, pad)))
    dstp = jnp.pad(dst.reshape(NW, EPW), ((0, 0), (0, pad)))
    valp = jnp.pad(vals.reshape(NW, EPW), ((0, 0), (0, pad)))
    meta = jnp.stack([srcp.reshape(NW, NCHP, CH),
                      dstp.reshape(NW, NCHP, CH)], axis=2)
    return meta.reshape(NW * NCHP, 2, CH), valp.reshape(NW * NCHP, CH)


def kernel(x, edge_index, edge_vals, W1, b1, W2, b2, W3, b3):
    meta, valp = _pack_meta(edge_index[0], edge_index[1], edge_vals)

    s1 = _tc_mm(x, W1)
    p1 = _sc_spmm(s1, meta, valp)
    s2 = _tc_combine_mm(p1, b1, W2)
    p2 = _sc_spmm(s2, meta, valp)
    s3 = _tc_combine_mm(p2, b2, W3)
    p3 = _sc_spmm(s3, meta, valp)
    return _tc_softmax(p3, b3)[:N]


# speedup vs baseline: 1.2398x; 1.2398x over previous
import functools

import jax
import jax.numpy as jnp
from jax import lax
from jax.experimental import pallas as pl
from jax.experimental.pallas import tpu as pltpu
from jax.experimental.pallas import tpu_sc as plsc

N = 10000
D = 128
H = 128
C = 16
E = 320000

NC = 2
NS = 16
NW = NC * NS
EPW = E // NW
CH = 80
NCH = EPW // CH
NP = 10240
RPT = NP // NS
SL = 4


def _row_block(rows):
    return 1000 if rows % 1000 == 0 else 1024


def _mm_body(x_ref, w_ref, o_ref):
    o_ref[...] = jnp.dot(x_ref[...], w_ref[...],
                         preferred_element_type=jnp.float32)


def _tc_mm(x, w):
    rows, h2 = x.shape[0], w.shape[1]
    mb = _row_block(rows)
    return pl.pallas_call(
        _mm_body,
        grid=(rows // mb,),
        in_specs=[pl.BlockSpec((mb, x.shape[1]), lambda i: (i, 0)),
                  pl.BlockSpec((x.shape[1], h2), lambda i: (0, 0))],
        out_specs=pl.BlockSpec((mb, h2), lambda i: (i, 0)),
        out_shape=jax.ShapeDtypeStruct((rows, h2), jnp.float32),
    )(x, w)


def _combine_mm_body(p_ref, b_ref, w_ref, o_ref):
    h = jnp.maximum(p_ref[0] + p_ref[1] + b_ref[...], 0.0)
    o_ref[...] = jnp.dot(h, w_ref[...], preferred_element_type=jnp.float32)


def _tc_combine_mm(p, b, w):
    rows, f = p.shape[1], p.shape[2]
    h2 = w.shape[1]
    mb = _row_block(rows)
    return pl.pallas_call(
        _combine_mm_body,
        grid=(rows // mb,),
        in_specs=[pl.BlockSpec((2, mb, f), lambda i: (0, i, 0)),
                  pl.BlockSpec((1, f), lambda i: (0, 0)),
                  pl.BlockSpec((f, h2), lambda i: (0, 0))],
        out_specs=pl.BlockSpec((mb, h2), lambda i: (i, 0)),
        out_shape=jax.ShapeDtypeStruct((rows, h2), jnp.float32),
    )(p, b.reshape(1, f), w)


def _softmax_body(p_ref, b_ref, o_ref):
    z = p_ref[0] + p_ref[1] + b_ref[...]
    z = z - jnp.max(z, axis=-1, keepdims=True)
    e = jnp.exp(z)
    o_ref[...] = e / jnp.sum(e, axis=-1, keepdims=True)


def _tc_softmax(p, b):
    rows, f = p.shape[1], p.shape[2]
    mb = _row_block(rows)
    return pl.pallas_call(
        _softmax_body,
        grid=(rows // mb,),
        in_specs=[pl.BlockSpec((2, mb, f), lambda i: (0, i, 0)),
                  pl.BlockSpec((1, f), lambda i: (0, 0))],
        out_specs=pl.BlockSpec((mb, f), lambda i: (i, 0)),
        out_shape=jax.ShapeDtypeStruct((rows, f), jnp.float32),
    )(p, b.reshape(1, f))


def _spmm_body(f, m_hbm, src_hbm, dst_hbm, val_hbm, out_hbm,
               src0, src1, src2, src3, dst0, dst1, dst2, dst3,
               val0, val1, val2, val3, rows0, rows1, rows2, rows3, acc,
               psem0, psem1, psem2, psem3, gsem0, gsem1, gsem2, gsem3,
               ssem0, ssem1, ssem2, ssem3):
    c = lax.axis_index("c")
    s = lax.axis_index("s")
    w = c * NS + s
    base = w * EPW

    srcv = (src0, src1, src2, src3)
    dstv = (dst0, dst1, dst2, dst3)
    valv = (val0, val1, val2, val3)
    rows = (rows0, rows1, rows2, rows3)
    psem = (psem0, psem1, psem2, psem3)
    gsem = (gsem0, gsem1, gsem2, gsem3)
    ssem = (ssem0, ssem1, ssem2, ssem3)

    zvec = jnp.zeros((16,), jnp.float32)

    def zrow(r, _):
        for t in range(f // 16):
            rows0[r, pl.ds(16 * t, 16)] = zvec
        return ()

    lax.fori_loop(0, CH, zrow, ())
    for i in range(RPT // CH):
        pltpu.sync_copy(rows0, acc.at[pl.ds(s * RPT + i * CH, CH)])
    plsc.subcore_barrier()

    def start_small(i, b):
        off = base + i * CH
        pltpu.async_copy(src_hbm.at[pl.ds(off, CH)], srcv[b], psem[b])
        pltpu.async_copy(dst_hbm.at[pl.ds(off, CH)], dstv[b], psem[b])
        pltpu.async_copy(val_hbm.at[pl.ds(off, CH)], valv[b], psem[b])

    def wait_small(i, b):
        off = base + i * CH
        pltpu.make_async_copy(src_hbm.at[pl.ds(off, CH)], srcv[b],
                              psem[b]).wait()
        pltpu.make_async_copy(dst_hbm.at[pl.ds(off, CH)], dstv[b],
                              psem[b]).wait()
        pltpu.make_async_copy(val_hbm.at[pl.ds(off, CH)], valv[b],
                              psem[b]).wait()

    def start_gather(b):
        pltpu.async_copy(m_hbm.at[srcv[b]], rows[b], gsem[b])

    def wait_gather(b):
        pltpu.make_async_copy(m_hbm.at[srcv[b]], rows[b], gsem[b]).wait()

    def start_scatter(b):
        pltpu.async_copy(rows[b], acc.at[dstv[b]], ssem[b], add=True)

    def wait_scatter(b):
        pltpu.make_async_copy(rows[b], acc.at[dstv[b]], ssem[b]).wait()

    def step(i, b):
        @pl.when(i >= 2)
        def _():
            wait_scatter((b + 2) % SL)

        @pl.when(i + 2 < NCH)
        def _():
            start_small(i + 2, (b + 2) % SL)

        @pl.when(i + 1 < NCH)
        def _():
            wait_small(i + 1, (b + 1) % SL)
            start_gather((b + 1) % SL)

        wait_gather(b)

        def scale(g, _):
            vv = valv[b][pl.ds(16 * g, 16)]
            for jj in range(16):
                j = 16 * g + jj
                bv = lax.broadcast(vv[jj], (16,))
                for t in range(f // 16):
                    sl = pl.ds(16 * t, 16)
                    rows[b][j, sl] = rows[b][j, sl] * bv
            return ()

        lax.fori_loop(0, CH // 16, scale, ())

        start_scatter(b)

    start_small(0, 0)
    start_small(1, 1)
    wait_small(0, 0)
    start_gather(0)

    def chunk(i, _):
        for b in range(SL):
            @pl.when(i % SL == b)
            def _(b=b):
                step(i, b)
        return ()

    lax.fori_loop(0, NCH, chunk, ())
    wait_scatter((NCH - 2) % SL)
    wait_scatter((NCH - 1) % SL)
    plsc.subcore_barrier()

    pltpu.sync_copy(acc.at[pl.ds(s * RPT, RPT)],
                    out_hbm.at[c, pl.ds(s * RPT, RPT)])


def _sc_spmm(m, src, dst, vals):
    f = m.shape[1]
    mesh = plsc.VectorSubcoreMesh(core_axis_name="c", subcore_axis_name="s",
                                  num_cores=NC, num_subcores=NS)
    return pl.kernel(
        functools.partial(_spmm_body, f),
        out_type=jax.ShapeDtypeStruct((2, NP, f), jnp.float32),
        mesh=mesh,
        compiler_params=pltpu.CompilerParams(use_tc_tiling_on_sc=False),
        scratch_types=(
            [pltpu.VMEM((CH,), jnp.int32) for _ in range(SL)]
            + [pltpu.VMEM((CH,), jnp.int32) for _ in range(SL)]
            + [pltpu.VMEM((CH,), jnp.float32) for _ in range(SL)]
            + [pltpu.VMEM((CH, f), jnp.float32) for _ in range(SL)]
            + [pltpu.VMEM_SHARED((NP, f), jnp.float32)]
            + [pltpu.SemaphoreType.DMA for _ in range(3 * SL)]
        ),
    )(m, src, dst, vals)


def kernel(x, edge_index, edge_vals, W1, b1, W2, b2, W3, b3):
    src = edge_index[0]
    dst = edge_index[1]

    s1 = _tc_mm(x, W1)
    p1 = _sc_spmm(s1, src, dst, edge_vals)
    s2 = _tc_combine_mm(p1, b1, W2)
    p2 = _sc_spmm(s2, src, dst, edge_vals)
    s3 = _tc_combine_mm(p2, b2, W3)
    p3 = _sc_spmm(s3, src, dst, edge_vals)
    return _tc_softmax(p3, b3)[:N]


# speedup vs baseline: 1.3284x; 1.0714x over previous
import functools

import jax
import jax.numpy as jnp
from jax import lax
from jax.experimental import pallas as pl
from jax.experimental.pallas import tpu as pltpu
from jax.experimental.pallas import tpu_sc as plsc

N = 10000
D = 128
H = 128
C = 16
E = 320000

NC = 2
NS = 16
NW = NC * NS
EPW = E // NW
CH = 80
NCH = EPW // CH
NP = 10240
RPT = NP // NS
SL = 4


def _row_block(rows):
    return 1000 if rows % 1000 == 0 else 1024


def _mm_body(x_ref, w_ref, o_ref):
    o_ref[...] = jnp.dot(x_ref[...], w_ref[...],
                         preferred_element_type=jnp.float32)


def _tc_mm(x, w):
    rows, h2 = x.shape[0], w.shape[1]
    mb = _row_block(rows)
    return pl.pallas_call(
        _mm_body,
        grid=(rows // mb,),
        in_specs=[pl.BlockSpec((mb, x.shape[1]), lambda i: (i, 0)),
                  pl.BlockSpec((x.shape[1], h2), lambda i: (0, 0))],
        out_specs=pl.BlockSpec((mb, h2), lambda i: (i, 0)),
        out_shape=jax.ShapeDtypeStruct((rows, h2), jnp.float32),
    )(x, w)


def _combine_mm_body(p_ref, b_ref, w_ref, o_ref):
    h = jnp.maximum(p_ref[0] + p_ref[1] + b_ref[...], 0.0)
    o_ref[...] = jnp.dot(h, w_ref[...], preferred_element_type=jnp.float32)


def _tc_combine_mm(p, b, w):
    rows, f = p.shape[1], p.shape[2]
    h2 = w.shape[1]
    mb = _row_block(rows)
    return pl.pallas_call(
        _combine_mm_body,
        grid=(rows // mb,),
        in_specs=[pl.BlockSpec((2, mb, f), lambda i: (0, i, 0)),
                  pl.BlockSpec((1, f), lambda i: (0, 0)),
                  pl.BlockSpec((f, h2), lambda i: (0, 0))],
        out_specs=pl.BlockSpec((mb, h2), lambda i: (i, 0)),
        out_shape=jax.ShapeDtypeStruct((rows, h2), jnp.float32),
    )(p, b.reshape(1, f), w)


def _softmax_body(p_ref, b_ref, o_ref):
    z = p_ref[0] + p_ref[1] + b_ref[...]
    z = z - jnp.max(z, axis=-1, keepdims=True)
    e = jnp.exp(z)
    o_ref[...] = e / jnp.sum(e, axis=-1, keepdims=True)


def _tc_softmax(p, b):
    rows, f = p.shape[1], p.shape[2]
    mb = _row_block(rows)
    return pl.pallas_call(
        _softmax_body,
        grid=(rows // mb,),
        in_specs=[pl.BlockSpec((2, mb, f), lambda i: (0, i, 0)),
                  pl.BlockSpec((1, f), lambda i: (0, 0))],
        out_specs=pl.BlockSpec((mb, f), lambda i: (i, 0)),
        out_shape=jax.ShapeDtypeStruct((rows, f), jnp.float32),
    )(p, b.reshape(1, f))


def _spmm_body(f, m_hbm, src_hbm, dst_hbm, val_hbm, out_hbm,
               src0, src1, src2, src3, dst0, dst1, dst2, dst3,
               val0, val1, val2, val3, sd0, sd1, sd2, sd3,
               rows0, rows1, rows2, rows3, acc,
               psem0, psem1, psem2, psem3, gsem0, gsem1, gsem2, gsem3,
               ssem0, ssem1, ssem2, ssem3):
    c = lax.axis_index("c")
    s = lax.axis_index("s")
    w = c * NS + s
    base = w * EPW

    srcv = (src0, src1, src2, src3)
    dstv = (dst0, dst1, dst2, dst3)
    valv = (val0, val1, val2, val3)
    sdst = (sd0, sd1, sd2, sd3)
    rows = (rows0, rows1, rows2, rows3)
    psem = (psem0, psem1, psem2, psem3)
    gsem = (gsem0, gsem1, gsem2, gsem3)
    ssem = (ssem0, ssem1, ssem2, ssem3)

    zvec = jnp.zeros((16,), jnp.float32)

    def zrow(r, _):
        for t in range(f // 16):
            rows0[r, pl.ds(16 * t, 16)] = zvec
        return ()

    lax.fori_loop(0, CH, zrow, ())
    for i in range(RPT // CH):
        pltpu.sync_copy(rows0, acc.at[pl.ds(s * RPT + i * CH, CH)])
    plsc.subcore_barrier()

    def start_small(i, b):
        off = base + i * CH
        pltpu.async_copy(src_hbm.at[pl.ds(off, CH)], srcv[b], psem[b])
        pltpu.async_copy(dst_hbm.at[pl.ds(off, CH)], dstv[b], psem[b])
        pltpu.async_copy(val_hbm.at[pl.ds(off, CH)], valv[b], psem[b])

    def wait_small(i, b):
        off = base + i * CH
        pltpu.make_async_copy(src_hbm.at[pl.ds(off, CH)], srcv[b],
                              psem[b]).wait()
        pltpu.make_async_copy(dst_hbm.at[pl.ds(off, CH)], dstv[b],
                              psem[b]).wait()
        pltpu.make_async_copy(val_hbm.at[pl.ds(off, CH)], valv[b],
                              psem[b]).wait()

    def start_gather(b):
        pltpu.async_copy(m_hbm.at[srcv[b]], rows[b], gsem[b])

    def wait_gather(b):
        pltpu.make_async_copy(m_hbm.at[srcv[b]], rows[b], gsem[b]).wait()

    def start_scatter(b):
        pltpu.async_copy(rows[b], acc.at[sdst[b]], ssem[b], add=True)

    def wait_scatter(b):
        pltpu.make_async_copy(rows[b], acc.at[sdst[b]], ssem[b]).wait()

    def step(i, b):
        @pl.when(i >= 2)
        def _():
            wait_scatter((b + 2) % SL)

        @pl.when(i + 3 < NCH)
        def _():
            start_small(i + 3, (b + 3) % SL)

        @pl.when(i + 2 < NCH)
        def _():
            wait_small(i + 2, (b + 2) % SL)
            start_gather((b + 2) % SL)

        wait_gather(b)

        for k in range(CH // 16):
            sdst[b][pl.ds(16 * k, 16)] = dstv[b][pl.ds(16 * k, 16)]

        def scale(g, _):
            vv = valv[b][pl.ds(16 * g, 16)]
            for jj in range(16):
                j = 16 * g + jj
                bv = lax.broadcast(vv[jj], (16,))
                for t in range(f // 16):
                    sl = pl.ds(16 * t, 16)
                    rows[b][j, sl] = rows[b][j, sl] * bv
            return ()

        lax.fori_loop(0, CH // 16, scale, ())

        start_scatter(b)

    start_small(0, 0)
    start_small(1, 1)
    start_small(2, 2)
    wait_small(0, 0)
    start_gather(0)
    wait_small(1, 1)
    start_gather(1)

    def chunk(i, _):
        for b in range(SL):
            @pl.when(i % SL == b)
            def _(b=b):
                step(i, b)
        return ()

    lax.fori_loop(0, NCH, chunk, ())
    wait_scatter((NCH - 2) % SL)
    wait_scatter((NCH - 1) % SL)
    plsc.subcore_barrier()

    pltpu.sync_copy(acc.at[pl.ds(s * RPT, RPT)],
                    out_hbm.at[c, pl.ds(s * RPT, RPT)])


def _sc_spmm(m, src, dst, vals):
    f = m.shape[1]
    mesh = plsc.VectorSubcoreMesh(core_axis_name="c", subcore_axis_name="s",
                                  num_cores=NC, num_subcores=NS)
    return pl.kernel(
        functools.partial(_spmm_body, f),
        out_type=jax.ShapeDtypeStruct((2, NP, f), jnp.float32),
        mesh=mesh,
        compiler_params=pltpu.CompilerParams(use_tc_tiling_on_sc=False),
        scratch_types=(
            [pltpu.VMEM((CH,), jnp.int32) for _ in range(SL)]
            + [pltpu.VMEM((CH,), jnp.int32) for _ in range(SL)]
            + [pltpu.VMEM((CH,), jnp.float32) for _ in range(SL)]
            + [pltpu.VMEM((CH,), jnp.int32) for _ in range(SL)]
            + [pltpu.VMEM((CH, f), jnp.float32) for _ in range(SL)]
            + [pltpu.VMEM_SHARED((NP, f), jnp.float32)]
            + [pltpu.SemaphoreType.DMA for _ in range(3 * SL)]
        ),
    )(m, src, dst, vals)


def kernel(x, edge_index, edge_vals, W1, b1, W2, b2, W3, b3):
    src = edge_index[0]
    dst = edge_index[1]

    s1 = _tc_mm(x, W1)
    p1 = _sc_spmm(s1, src, dst, edge_vals)
    s2 = _tc_combine_mm(p1, b1, W2)
    p2 = _sc_spmm(s2, src, dst, edge_vals)
    s3 = _tc_combine_mm(p2, b2, W3)
    p3 = _sc_spmm(s3, src, dst, edge_vals)
    return _tc_softmax(p3, b3)[:N]
